# baseline (speedup 1.0000x reference)
import functools

import jax
import jax.numpy as jnp
from jax.experimental import pallas as pl
from jax.experimental.pallas import tpu as pltpu

_LOW = 0.0
_HIGH = 1.0


def _body(xc_ref, xn_ref, pw_ref, lh_ref, out_ref, *, n_cat, n_num, levels, size):
    xc = xc_ref[...]
    pw = pw_ref[...]
    cat_counts = jnp.sum(jnp.bitwise_xor(xc, pw[None, :n_cat, :]), axis=1)

    xn = xn_ref[...]
    idx = jnp.clip(
        jnp.round((xn - _LOW) / (_HIGH - _LOW) * (levels - 1)), 0, levels - 1
    ).astype(jnp.int32)
    bb = xn.shape[0]
    oh = (idx[:, :, None]
          == jax.lax.broadcasted_iota(jnp.int32, (bb, n_num, levels), 2)
          ).astype(jnp.float32)
    lh = lh_ref[...].astype(jnp.float32)
    num_hv = jax.lax.dot_general(
        oh, lh, (((2,), (0,)), ((), ())), preferred_element_type=jnp.float32
    ).astype(jnp.int32)
    num_counts = jnp.sum(jnp.bitwise_xor(num_hv, pw[None, n_cat:, :]), axis=1)

    total = cat_counts + num_counts
    out_ref[...] = (total * 2 >= size).astype(jnp.int32)


@jax.jit
def kernel(x_categorical, x_numeric, position_weight, level_hvs):
    b, n_cat, d = x_categorical.shape
    n_num = x_numeric.shape[1]
    size = position_weight.shape[0]
    levels = level_hvs.shape[0]
    bb = 8
    grid = (b // bb,)
    body = functools.partial(
        _body, n_cat=n_cat, n_num=n_num, levels=levels, size=size)
    return pl.pallas_call(
        body,
        grid=grid,
        in_specs=[
            pl.BlockSpec((bb, n_cat, d), lambda i: (i, 0, 0)),
            pl.BlockSpec((bb, n_num), lambda i: (i, 0)),
            pl.BlockSpec((size, d), lambda i: (0, 0)),
            pl.BlockSpec((levels, d), lambda i: (0, 0)),
        ],
        out_specs=pl.BlockSpec((bb, d), lambda i: (i, 0)),
        out_shape=jax.ShapeDtypeStruct((b, d), jnp.int32),
        compiler_params=pltpu.CompilerParams(
            dimension_semantics=("parallel",),
        ),
    )(x_categorical, x_numeric, position_weight, level_hvs)
